# baseline (speedup 1.0000x reference)
import functools

import jax
import jax.numpy as jnp
from jax import lax
from jax.experimental import pallas as pl
from jax.experimental.pallas import tpu as pltpu
from jax.experimental.pallas import tpu_sc as plsc

B, S = 16, 512
N = B * S
H = 768
GLYPH_DIM = 1728
VOCAB = 23236
P_LOCS = 8
P_VOCAB = 32

NUM_CORES = 2
NUM_SUBCORES = 16
NW = NUM_CORES * NUM_SUBCORES
TPW = N // NW
CHUNK = 64


def _sc_gather_rows(ids, table):
    mesh = plsc.VectorSubcoreMesh(core_axis_name="c", subcore_axis_name="s")

    @functools.partial(
        pl.kernel,
        out_type=jax.ShapeDtypeStruct((N, H), jnp.float32),
        mesh=mesh,
        scratch_types=[
            pltpu.VMEM((CHUNK,), jnp.int32),
            pltpu.VMEM((CHUNK, H), jnp.float32),
            pltpu.SemaphoreType.DMA,
        ],
    )
    def k(ids_hbm, table_hbm, out_hbm, idx_v, rows_v, sem):
        wid = lax.axis_index("s") * NUM_CORES + lax.axis_index("c")
        base = wid * TPW
        for c in range(TPW // CHUNK):
            off = base + c * CHUNK
            pltpu.sync_copy(ids_hbm.at[pl.ds(off, CHUNK)], idx_v)
            pltpu.async_copy(table_hbm.at[idx_v], rows_v, sem).wait()
            pltpu.sync_copy(rows_v, out_hbm.at[pl.ds(off, CHUNK)])

    return k(ids, table)


def _sc_gather_word_type_pos(ids, word_table, type_table, pos_table):
    mesh = plsc.VectorSubcoreMesh(core_axis_name="c", subcore_axis_name="s")
    POS_PW = S // NW

    @functools.partial(
        pl.kernel,
        out_type=[
            jax.ShapeDtypeStruct((N, H), jnp.float32),
            jax.ShapeDtypeStruct((N, H), jnp.float32),
            jax.ShapeDtypeStruct((S, H), jnp.float32),
        ],
        mesh=mesh,
        scratch_types=[
            pltpu.VMEM((CHUNK,), jnp.int32),
            pltpu.VMEM((CHUNK, H), jnp.float32),
            pltpu.VMEM((CHUNK, H), jnp.float32),
            pltpu.VMEM((POS_PW, H), jnp.float32),
            pltpu.SemaphoreType.DMA,
            pltpu.SemaphoreType.DMA,
        ],
    )
    def k(ids_hbm, word_hbm, type_hbm, pos_hbm, wout_hbm, tout_hbm, pout_hbm,
          idx_v, rows_v, trow_v, prow_v, sem, sem2):
        wid = lax.axis_index("s") * NUM_CORES + lax.axis_index("c")
        base = wid * TPW
        for j in range(CHUNK // 16):
            idx_v[pl.ds(j * 16, 16)] = jnp.zeros((16,), jnp.int32)
        pltpu.async_copy(type_hbm.at[idx_v], trow_v, sem2).wait()
        pltpu.sync_copy(pos_hbm.at[pl.ds(wid * POS_PW, POS_PW)], prow_v)
        pltpu.sync_copy(prow_v, pout_hbm.at[pl.ds(wid * POS_PW, POS_PW)])
        for c in range(TPW // CHUNK):
            off = base + c * CHUNK
            pltpu.sync_copy(ids_hbm.at[pl.ds(off, CHUNK)], idx_v)
            pltpu.async_copy(word_hbm.at[idx_v], rows_v, sem).wait()
            pltpu.sync_copy(rows_v, wout_hbm.at[pl.ds(off, CHUNK)])
            pltpu.sync_copy(trow_v, tout_hbm.at[pl.ds(off, CHUNK)])

    return k(ids, word_table, type_table, pos_table)


GM = 1024
G_STEPS = (VOCAB + GM - 1) // GM


def _pg_body(gt_ref, gw_ref, gb_ref, out_ref):
    dn = (((0,), (0,)), ((), ()))
    x = gt_ref[...].astype(jnp.bfloat16)
    acc = lax.dot_general(x, gw_ref[...], dn,
                          preferred_element_type=jnp.float32)
    out_ref[...] = acc + gb_ref[...]


def _pg_matmul(glyph_table_t, glyph_w_t_bf, glyph_b):
    return pl.pallas_call(
        _pg_body,
        grid=(G_STEPS,),
        in_specs=[
            pl.BlockSpec((GLYPH_DIM, GM), lambda i: (0, i)),
            pl.BlockSpec((GLYPH_DIM, H), lambda i: (0, 0)),
            pl.BlockSpec((1, H), lambda i: (0, 0)),
        ],
        compiler_params=pltpu.CompilerParams(
            dimension_semantics=("arbitrary",),
            fuse_transposed_lhs_in_matmul=True),
        out_specs=pl.BlockSpec((GM, H), lambda i: (i, 0)),
        out_shape=jax.ShapeDtypeStruct((VOCAB, H), jnp.float32),
    )(glyph_table_t, glyph_w_t_bf, glyph_b)


MB = 512
GRID = N // MB


def _tc_body(ids_ref, pt_ref, w0_ref, w1_ref, cb_ref, pout_ref):
    f32 = jnp.float32
    dn = (((1,), (1,)), ((), ()))

    p0 = lax.dot_general(pt_ref[...], w0_ref[...], dn,
                         preferred_element_type=f32).astype(jnp.bfloat16)
    p1 = lax.dot_general(pt_ref[...], w1_ref[...], dn,
                         preferred_element_type=f32).astype(jnp.bfloat16)
    ids = ids_ref[...]
    lanes = lax.broadcasted_iota(jnp.int32, (MB, 2 * P_VOCAB), 1)
    base = lanes & (P_VOCAB - 1)
    is_hi = lanes >= P_VOCAB
    p01 = jnp.concatenate([p0, p1], axis=0)
    dn0 = (((1,), (0,)), ((), ()))
    acc = None
    for l in range(P_LOCS - 1):
        sel = jnp.where(is_hi, ids[:, l + 1][:, None], ids[:, l][:, None])
        ohp = (base == sel).astype(jnp.bfloat16)
        e = lax.dot_general(ohp, p01, dn0, preferred_element_type=f32)
        acc = e if acc is None else jnp.maximum(acc, e)
    pout_ref[...] = acc + cb_ref[...]


def _tc_fused(pids, ptab_bf, w0_bf, w1_bf, conv_b):
    f32 = jnp.float32
    return pl.pallas_call(
        _tc_body,
        grid=(GRID,),
        in_specs=[
            pl.BlockSpec((MB, P_LOCS), lambda i: (i, 0)),
            pl.BlockSpec((P_VOCAB, 128), lambda i: (0, 0)),
            pl.BlockSpec((H, 128), lambda i: (0, 0)),
            pl.BlockSpec((H, 128), lambda i: (0, 0)),
            pl.BlockSpec((1, H), lambda i: (0, 0)),
        ],
        out_specs=pl.BlockSpec((MB, H), lambda i: (i, 0)),
        out_shape=jax.ShapeDtypeStruct((N, H), f32),
    )(pids, ptab_bf, w0_bf, w1_bf, conv_b)


def kernel(input_ids, pinyin_ids, word_table, pinyin_char_table, conv_w,
           conv_b, glyph_table, glyph_w, glyph_b, pos_table, type_table):
    ids = input_ids.reshape(N).astype(jnp.int32)
    bf = jnp.bfloat16

    word_emb, type_emb, pos_emb = _sc_gather_word_type_pos(
        ids, word_table, type_table, pos_table)
    pg = _pg_matmul(glyph_table.T, glyph_w.T.astype(bf),
                    glyph_b.reshape(1, H))
    glyph_emb = _sc_gather_rows(ids, pg)

    pids = pinyin_ids.reshape(N, P_LOCS).astype(jnp.int32)
    pinyin_emb = _tc_fused(
        pids,
        pinyin_char_table.astype(bf),
        conv_w[:, :, 0].astype(bf),
        conv_w[:, :, 1].astype(bf),
        conv_b.reshape(1, H),
    )

    return (
        word_emb.reshape(B, S, H),
        pinyin_emb.reshape(B, S, H),
        glyph_emb.reshape(B, S, H),
        pos_emb.reshape(1, S, H),
        type_emb.reshape(B, S, H),
    )


# speedup vs baseline: 1.2170x; 1.2170x over previous
import functools

import jax
import jax.numpy as jnp
from jax import lax
from jax.experimental import pallas as pl
from jax.experimental.pallas import tpu as pltpu
from jax.experimental.pallas import tpu_sc as plsc

B, S = 16, 512
N = B * S
H = 768
GLYPH_DIM = 1728
VOCAB = 23236
P_LOCS = 8
P_VOCAB = 32

NUM_CORES = 2
NUM_SUBCORES = 16
NW = NUM_CORES * NUM_SUBCORES
TPW = N // NW
CHUNK = 64


def _sc_gather_rows(ids, table):
    mesh = plsc.VectorSubcoreMesh(core_axis_name="c", subcore_axis_name="s")

    @functools.partial(
        pl.kernel,
        out_type=jax.ShapeDtypeStruct((N, H), jnp.float32),
        mesh=mesh,
        scratch_types=[
            pltpu.VMEM((CHUNK,), jnp.int32),
            pltpu.VMEM((CHUNK, H), jnp.float32),
            pltpu.SemaphoreType.DMA,
        ],
    )
    def k(ids_hbm, table_hbm, out_hbm, idx_v, rows_v, sem):
        wid = lax.axis_index("s") * NUM_CORES + lax.axis_index("c")
        base = wid * TPW
        for c in range(TPW // CHUNK):
            off = base + c * CHUNK
            pltpu.sync_copy(ids_hbm.at[pl.ds(off, CHUNK)], idx_v)
            pltpu.async_copy(table_hbm.at[idx_v], rows_v, sem).wait()
            pltpu.sync_copy(rows_v, out_hbm.at[pl.ds(off, CHUNK)])

    return k(ids, table)


GM = 1024
G_STEPS = (VOCAB + GM - 1) // GM


def _pg_body(gt_ref, gw_ref, gb_ref, out_ref):
    dn = (((0,), (0,)), ((), ()))
    x = gt_ref[...].astype(jnp.bfloat16)
    acc = lax.dot_general(x, gw_ref[...], dn,
                          preferred_element_type=jnp.float32)
    out_ref[...] = acc + gb_ref[...]


def _pg_matmul(glyph_table_t, glyph_w_t_bf, glyph_b):
    return pl.pallas_call(
        _pg_body,
        grid=(G_STEPS,),
        in_specs=[
            pl.BlockSpec((GLYPH_DIM, GM), lambda i: (0, i)),
            pl.BlockSpec((GLYPH_DIM, H), lambda i: (0, 0)),
            pl.BlockSpec((1, H), lambda i: (0, 0)),
        ],
        compiler_params=pltpu.CompilerParams(
            dimension_semantics=("arbitrary",),
            fuse_transposed_lhs_in_matmul=True),
        out_specs=pl.BlockSpec((GM, H), lambda i: (i, 0)),
        out_shape=jax.ShapeDtypeStruct((VOCAB, H), jnp.float32),
    )(glyph_table_t, glyph_w_t_bf, glyph_b)


MB = 512
GRID = N // MB


def _tc_body(ids_ref, pt_ref, w0_ref, w1_ref, cb_ref, tt_ref,
             pout_ref, tout_ref):
    f32 = jnp.float32
    dn = (((1,), (1,)), ((), ()))

    p0 = lax.dot_general(pt_ref[...], w0_ref[...], dn,
                         preferred_element_type=f32).astype(jnp.bfloat16)
    p1 = lax.dot_general(pt_ref[...], w1_ref[...], dn,
                         preferred_element_type=f32).astype(jnp.bfloat16)
    ids = ids_ref[...]
    lanes = lax.broadcasted_iota(jnp.int32, (MB, 2 * P_VOCAB), 1)
    base = lanes & (P_VOCAB - 1)
    is_hi = lanes >= P_VOCAB
    p01 = jnp.concatenate([p0, p1], axis=0)
    dn0 = (((1,), (0,)), ((), ()))
    acc = None
    for l in range(P_LOCS - 1):
        sel = jnp.where(is_hi, ids[:, l + 1][:, None], ids[:, l][:, None])
        ohp = (base == sel).astype(jnp.bfloat16)
        e = lax.dot_general(ohp, p01, dn0, preferred_element_type=f32)
        acc = e if acc is None else jnp.maximum(acc, e)
    pout_ref[...] = acc + cb_ref[...]

    tout_ref[...] = jnp.broadcast_to(tt_ref[0:1, :], (MB, H))


def _tc_fused(pids, ptab_bf, w0_bf, w1_bf, conv_b, type_table):
    f32 = jnp.float32
    return pl.pallas_call(
        _tc_body,
        grid=(GRID,),
        in_specs=[
            pl.BlockSpec((MB, P_LOCS), lambda i: (i, 0)),
            pl.BlockSpec((P_VOCAB, 128), lambda i: (0, 0)),
            pl.BlockSpec((H, 128), lambda i: (0, 0)),
            pl.BlockSpec((H, 128), lambda i: (0, 0)),
            pl.BlockSpec((1, H), lambda i: (0, 0)),
            pl.BlockSpec((2, H), lambda i: (0, 0)),
        ],
        out_specs=[
            pl.BlockSpec((MB, H), lambda i: (i, 0)),
            pl.BlockSpec((MB, H), lambda i: (i, 0)),
        ],
        out_shape=[
            jax.ShapeDtypeStruct((N, H), f32),
            jax.ShapeDtypeStruct((N, H), f32),
        ],
    )(pids, ptab_bf, w0_bf, w1_bf, conv_b, type_table)


def kernel(input_ids, pinyin_ids, word_table, pinyin_char_table, conv_w,
           conv_b, glyph_table, glyph_w, glyph_b, pos_table, type_table):
    ids = input_ids.reshape(N).astype(jnp.int32)
    bf = jnp.bfloat16

    word_emb = _sc_gather_rows(ids, word_table)
    pg = _pg_matmul(glyph_table.T, glyph_w.T.astype(bf),
                    glyph_b.reshape(1, H))
    glyph_emb = _sc_gather_rows(ids, pg)

    pids = pinyin_ids.reshape(N, P_LOCS).astype(jnp.int32)
    pinyin_emb, type_emb = _tc_fused(
        pids,
        pinyin_char_table.astype(bf),
        conv_w[:, :, 0].astype(bf),
        conv_w[:, :, 1].astype(bf),
        conv_b.reshape(1, H),
        type_table,
    )

    return (
        word_emb.reshape(B, S, H),
        pinyin_emb.reshape(B, S, H),
        glyph_emb.reshape(B, S, H),
        pos_table.reshape(1, S, H),
        type_emb.reshape(B, S, H),
    )


# speedup vs baseline: 1.5388x; 1.2644x over previous
import functools

import jax
import jax.numpy as jnp
from jax import lax
from jax.experimental import pallas as pl
from jax.experimental.pallas import tpu as pltpu
from jax.experimental.pallas import tpu_sc as plsc

B, S = 16, 512
N = B * S
H = 768
GLYPH_DIM = 1728
VOCAB = 23236
P_LOCS = 8
P_VOCAB = 32

NUM_CORES = 2
NUM_SUBCORES = 16
NW = NUM_CORES * NUM_SUBCORES
TPW = N // NW
CHUNK = 64


def _sc_gather_rows(ids, table):
    mesh = plsc.VectorSubcoreMesh(core_axis_name="c", subcore_axis_name="s")

    @functools.partial(
        pl.kernel,
        out_type=jax.ShapeDtypeStruct((N, H), jnp.float32),
        mesh=mesh,
        scratch_types=[
            pltpu.VMEM((CHUNK,), jnp.int32),
            pltpu.VMEM((CHUNK, H), jnp.float32),
            pltpu.SemaphoreType.DMA,
        ],
    )
    def k(ids_hbm, table_hbm, out_hbm, idx_v, rows_v, sem):
        wid = lax.axis_index("s") * NUM_CORES + lax.axis_index("c")
        base = wid * TPW
        for c in range(TPW // CHUNK):
            off = base + c * CHUNK
            pltpu.sync_copy(ids_hbm.at[pl.ds(off, CHUNK)], idx_v)
            pltpu.async_copy(table_hbm.at[idx_v], rows_v, sem).wait()
            pltpu.sync_copy(rows_v, out_hbm.at[pl.ds(off, CHUNK)])

    return k(ids, table)


GM = 1024
G_STEPS = (VOCAB + GM - 1) // GM


def _pg_body(gt_ref, gw_ref, gb_ref, out_ref):
    dn = (((0,), (0,)), ((), ()))
    x = gt_ref[...].astype(jnp.bfloat16)
    acc = lax.dot_general(x, gw_ref[...], dn,
                          preferred_element_type=jnp.float32)
    out_ref[...] = acc + gb_ref[...]


def _pg_matmul(glyph_table_t, glyph_w_t_bf, glyph_b):
    return pl.pallas_call(
        _pg_body,
        grid=(G_STEPS,),
        in_specs=[
            pl.BlockSpec((GLYPH_DIM, GM), lambda i: (0, i)),
            pl.BlockSpec((GLYPH_DIM, H), lambda i: (0, 0)),
            pl.BlockSpec((1, H), lambda i: (0, 0)),
        ],
        compiler_params=pltpu.CompilerParams(
            dimension_semantics=("arbitrary",),
            fuse_transposed_lhs_in_matmul=False),
        out_specs=pl.BlockSpec((GM, H), lambda i: (i, 0)),
        out_shape=jax.ShapeDtypeStruct((VOCAB, H), jnp.float32),
    )(glyph_table_t, glyph_w_t_bf, glyph_b)


MB = 512
GRID = N // MB


def _tc_body(ids_ref, pt_ref, w0_ref, w1_ref, cb_ref, tt_ref,
             pout_ref, tout_ref):
    f32 = jnp.float32
    dn = (((1,), (1,)), ((), ()))

    p0 = lax.dot_general(pt_ref[...], w0_ref[...], dn,
                         preferred_element_type=f32).astype(jnp.bfloat16)
    p1 = lax.dot_general(pt_ref[...], w1_ref[...], dn,
                         preferred_element_type=f32).astype(jnp.bfloat16)
    ids = ids_ref[...]
    lanes = lax.broadcasted_iota(jnp.int32, (MB, 2 * P_VOCAB), 1)
    base = lanes & (P_VOCAB - 1)
    is_hi = lanes >= P_VOCAB
    p01 = jnp.concatenate([p0, p1], axis=0)
    dn0 = (((1,), (0,)), ((), ()))
    acc = None
    for l in range(P_LOCS - 1):
        sel = jnp.where(is_hi, ids[:, l + 1][:, None], ids[:, l][:, None])
        ohp = (base == sel).astype(jnp.bfloat16)
        e = lax.dot_general(ohp, p01, dn0, preferred_element_type=f32)
        acc = e if acc is None else jnp.maximum(acc, e)
    pout_ref[...] = acc + cb_ref[...]

    tout_ref[...] = jnp.broadcast_to(tt_ref[0:1, :], (MB, H))


def _tc_fused(pids, ptab_bf, w0_bf, w1_bf, conv_b, type_table):
    f32 = jnp.float32
    return pl.pallas_call(
        _tc_body,
        grid=(GRID,),
        in_specs=[
            pl.BlockSpec((MB, P_LOCS), lambda i: (i, 0)),
            pl.BlockSpec((P_VOCAB, 128), lambda i: (0, 0)),
            pl.BlockSpec((H, 128), lambda i: (0, 0)),
            pl.BlockSpec((H, 128), lambda i: (0, 0)),
            pl.BlockSpec((1, H), lambda i: (0, 0)),
            pl.BlockSpec((2, H), lambda i: (0, 0)),
        ],
        out_specs=[
            pl.BlockSpec((MB, H), lambda i: (i, 0)),
            pl.BlockSpec((MB, H), lambda i: (i, 0)),
        ],
        out_shape=[
            jax.ShapeDtypeStruct((N, H), f32),
            jax.ShapeDtypeStruct((N, H), f32),
        ],
    )(pids, ptab_bf, w0_bf, w1_bf, conv_b, type_table)


def kernel(input_ids, pinyin_ids, word_table, pinyin_char_table, conv_w,
           conv_b, glyph_table, glyph_w, glyph_b, pos_table, type_table):
    ids = input_ids.reshape(N).astype(jnp.int32)
    bf = jnp.bfloat16

    word_emb = _sc_gather_rows(ids, word_table)
    pg = _pg_matmul(glyph_table.T, glyph_w.T.astype(bf),
                    glyph_b.reshape(1, H))
    glyph_emb = _sc_gather_rows(ids, pg)

    pids = pinyin_ids.reshape(N, P_LOCS).astype(jnp.int32)
    pinyin_emb, type_emb = _tc_fused(
        pids,
        pinyin_char_table.astype(bf),
        conv_w[:, :, 0].astype(bf),
        conv_w[:, :, 1].astype(bf),
        conv_b.reshape(1, H),
        type_table,
    )

    return (
        word_emb.reshape(B, S, H),
        pinyin_emb.reshape(B, S, H),
        glyph_emb.reshape(B, S, H),
        pos_table.reshape(1, S, H),
        type_emb.reshape(B, S, H),
    )


# speedup vs baseline: 1.5743x; 1.0231x over previous
import functools

import jax
import jax.numpy as jnp
from jax import lax
from jax.experimental import pallas as pl
from jax.experimental.pallas import tpu as pltpu
from jax.experimental.pallas import tpu_sc as plsc

B, S = 16, 512
N = B * S
H = 768
GLYPH_DIM = 1728
VOCAB = 23236
P_LOCS = 8
P_VOCAB = 32

NUM_CORES = 2
NUM_SUBCORES = 16
NW = NUM_CORES * NUM_SUBCORES
TPW = N // NW
CHUNK = 64


def _sc_gather_rows(ids, table):
    mesh = plsc.VectorSubcoreMesh(core_axis_name="c", subcore_axis_name="s")

    @functools.partial(
        pl.kernel,
        out_type=jax.ShapeDtypeStruct((N, H), jnp.float32),
        mesh=mesh,
        scratch_types=[
            pltpu.VMEM((2, CHUNK), jnp.int32),
            pltpu.VMEM((2, CHUNK, H), jnp.float32),
            pltpu.SemaphoreType.DMA,
            pltpu.SemaphoreType.DMA,
        ],
    )
    def k(ids_hbm, table_hbm, out_hbm, idx_v, rows_v, sem0, sem1):
        wid = lax.axis_index("s") * NUM_CORES + lax.axis_index("c")
        base = wid * TPW
        nch = TPW // CHUNK
        sems = (sem0, sem1)
        pltpu.sync_copy(ids_hbm.at[pl.ds(base, CHUNK)], idx_v.at[0])
        cps = [pltpu.async_copy(table_hbm.at[idx_v.at[0]], rows_v.at[0],
                                sems[0]), None]
        for c in range(nch):
            b = c & 1
            nb = 1 - b
            if c + 1 < nch:
                off_n = base + (c + 1) * CHUNK
                pltpu.sync_copy(ids_hbm.at[pl.ds(off_n, CHUNK)], idx_v.at[nb])
                cps[nb] = pltpu.async_copy(table_hbm.at[idx_v.at[nb]],
                                           rows_v.at[nb], sems[nb])
            cps[b].wait()
            pltpu.sync_copy(rows_v.at[b], out_hbm.at[pl.ds(base + c * CHUNK,
                                                           CHUNK)])

    return k(ids, table)


GM = 1024
G_STEPS = (VOCAB + GM - 1) // GM


def _pg_body(gt_ref, gw_ref, gb_ref, out_ref):
    dn = (((0,), (0,)), ((), ()))
    x = gt_ref[...].astype(jnp.bfloat16)
    acc = lax.dot_general(x, gw_ref[...], dn,
                          preferred_element_type=jnp.float32)
    out_ref[...] = acc + gb_ref[...]


def _pg_matmul(glyph_table_t, glyph_w_t_bf, glyph_b):
    return pl.pallas_call(
        _pg_body,
        grid=(G_STEPS,),
        in_specs=[
            pl.BlockSpec((GLYPH_DIM, GM), lambda i: (0, i)),
            pl.BlockSpec((GLYPH_DIM, H), lambda i: (0, 0)),
            pl.BlockSpec((1, H), lambda i: (0, 0)),
        ],
        compiler_params=pltpu.CompilerParams(
            dimension_semantics=("arbitrary",),
            fuse_transposed_lhs_in_matmul=False),
        out_specs=pl.BlockSpec((GM, H), lambda i: (i, 0)),
        out_shape=jax.ShapeDtypeStruct((VOCAB, H), jnp.float32),
    )(glyph_table_t, glyph_w_t_bf, glyph_b)


MB = 1024
GRID = N // MB


def _tc_body(ids_ref, pt_ref, w0_ref, w1_ref, cb_ref, tt_ref,
             pout_ref, tout_ref):
    f32 = jnp.float32
    dn = (((1,), (1,)), ((), ()))

    p0 = lax.dot_general(pt_ref[...], w0_ref[...], dn,
                         preferred_element_type=f32).astype(jnp.bfloat16)
    p1 = lax.dot_general(pt_ref[...], w1_ref[...], dn,
                         preferred_element_type=f32).astype(jnp.bfloat16)
    ids = ids_ref[...]
    lanes = lax.broadcasted_iota(jnp.int32, (MB, 2 * P_VOCAB), 1)
    base = lanes & (P_VOCAB - 1)
    is_hi = lanes >= P_VOCAB
    p01 = jnp.concatenate([p0, p1], axis=0)
    dn0 = (((1,), (0,)), ((), ()))
    acc = None
    for l in range(P_LOCS - 1):
        sel = jnp.where(is_hi, ids[:, l + 1][:, None], ids[:, l][:, None])
        ohp = (base == sel).astype(jnp.bfloat16)
        e = lax.dot_general(ohp, p01, dn0, preferred_element_type=f32)
        acc = e if acc is None else jnp.maximum(acc, e)
    pout_ref[...] = acc + cb_ref[...]

    tout_ref[...] = jnp.broadcast_to(tt_ref[0:1, :], (MB, H))


def _tc_fused(pids, ptab_bf, w0_bf, w1_bf, conv_b, type_table):
    f32 = jnp.float32
    return pl.pallas_call(
        _tc_body,
        grid=(GRID,),
        in_specs=[
            pl.BlockSpec((MB, P_LOCS), lambda i: (i, 0)),
            pl.BlockSpec((P_VOCAB, 128), lambda i: (0, 0)),
            pl.BlockSpec((H, 128), lambda i: (0, 0)),
            pl.BlockSpec((H, 128), lambda i: (0, 0)),
            pl.BlockSpec((1, H), lambda i: (0, 0)),
            pl.BlockSpec((2, H), lambda i: (0, 0)),
        ],
        out_specs=[
            pl.BlockSpec((MB, H), lambda i: (i, 0)),
            pl.BlockSpec((MB, H), lambda i: (i, 0)),
        ],
        out_shape=[
            jax.ShapeDtypeStruct((N, H), f32),
            jax.ShapeDtypeStruct((N, H), f32),
        ],
    )(pids, ptab_bf, w0_bf, w1_bf, conv_b, type_table)


def kernel(input_ids, pinyin_ids, word_table, pinyin_char_table, conv_w,
           conv_b, glyph_table, glyph_w, glyph_b, pos_table, type_table):
    ids = input_ids.reshape(N).astype(jnp.int32)
    bf = jnp.bfloat16

    word_emb = _sc_gather_rows(ids, word_table)
    pg = _pg_matmul(glyph_table.T, glyph_w.T.astype(bf),
                    glyph_b.reshape(1, H))
    glyph_emb = _sc_gather_rows(ids, pg)

    pids = pinyin_ids.reshape(N, P_LOCS).astype(jnp.int32)
    pinyin_emb, type_emb = _tc_fused(
        pids,
        pinyin_char_table.astype(bf),
        conv_w[:, :, 0].astype(bf),
        conv_w[:, :, 1].astype(bf),
        conv_b.reshape(1, H),
        type_table,
    )

    return (
        word_emb.reshape(B, S, H),
        pinyin_emb.reshape(B, S, H),
        glyph_emb.reshape(B, S, H),
        pos_table.reshape(1, S, H),
        type_emb.reshape(B, S, H),
    )
